# baseline (speedup 1.0000x reference)
import jax
import jax.numpy as jnp
from jax import lax
from jax.experimental import pallas as pl
from jax.experimental.pallas import tpu as pltpu
from jax.experimental.pallas import tpu_sc as plsc

NC, NS, L = 2, 16, 16
NW = NC * NS
B = 16384
BPW = B // NW
NCHUNK = 4
CHUNK = BPW // NCHUNK
D = 16
GPC = CHUNK // L


def _body(uidx_hbm, iidx_hbm, utab_hbm, itab_hbm, out_hbm,
          uidx_v, iidx_v, urows_v, irows_v, out_v,
          usem, isem):
    c = lax.axis_index("c")
    s = lax.axis_index("s")
    wid = s * NC + c

    pltpu.sync_copy(uidx_hbm.at[wid], uidx_v)
    pltpu.sync_copy(iidx_hbm.at[wid], iidx_v)

    lanes = lax.iota(jnp.int32, L)

    for k in range(NCHUNK):
        ucp = pltpu.async_copy(
            utab_hbm.at[uidx_v.at[pl.ds(k * CHUNK, CHUNK)]], urows_v, usem)
        icp = pltpu.async_copy(
            itab_hbm.at[iidx_v.at[pl.ds(k * CHUNK, CHUNK)]], irows_v, isem)
        ucp.wait()
        icp.wait()

        def group(g, carry):
            j0 = k * CHUNK + g * L
            rows = lanes + g * L
            acc = jnp.zeros((L,), jnp.float32)
            for d in range(D):
                dcol = jnp.full((L,), d, jnp.int32)
                uv = plsc.load_gather(urows_v, [rows, dcol])
                iv = plsc.load_gather(irows_v, [rows, dcol])
                acc = acc + uv * iv
            out_v[pl.ds(j0, L)] = 1.0 / (1.0 + jnp.exp(-acc))
            return carry

        lax.fori_loop(0, GPC, group, 0)

    pltpu.sync_copy(out_v, out_hbm.at[wid])


@jax.jit
def kernel(userIdx, itemIdx, uEmbed, iEmbed):
    uidx = userIdx.astype(jnp.int32).reshape(NW, BPW)
    iidx = itemIdx.astype(jnp.int32).reshape(NW, BPW)
    mesh = plsc.VectorSubcoreMesh(
        core_axis_name="c", subcore_axis_name="s",
        num_cores=NC, num_subcores=NS)
    out = pl.kernel(
        _body,
        out_type=jax.ShapeDtypeStruct((NW, BPW), jnp.float32),
        mesh=mesh,
        compiler_params=pltpu.CompilerParams(
            needs_layout_passes=False, use_tc_tiling_on_sc=False),
        scratch_types=[
            pltpu.VMEM((BPW,), jnp.int32),
            pltpu.VMEM((BPW,), jnp.int32),
            pltpu.VMEM((CHUNK, D), jnp.float32),
            pltpu.VMEM((CHUNK, D), jnp.float32),
            pltpu.VMEM((BPW,), jnp.float32),
            pltpu.SemaphoreType.DMA,
            pltpu.SemaphoreType.DMA,
        ],
    )(uidx, iidx, uEmbed, iEmbed)
    return out.reshape(-1)


# speedup vs baseline: 2.0439x; 2.0439x over previous
import jax
import jax.numpy as jnp
from jax import lax
from jax.experimental import pallas as pl
from jax.experimental.pallas import tpu as pltpu
from jax.experimental.pallas import tpu_sc as plsc

NC, NS, L = 2, 16, 16
NW = NC * NS
B = 16384
BPW = B // NW
NCHUNK = 4
CHUNK = BPW // NCHUNK
D = 16
RPS = 8
SUP = 128
GPC = CHUNK // L
N = 1000000
NSUP = N // RPS

WCOL = 896
NWIN_FULL = N // WCOL
TAIL = N - NWIN_FULL * WCOL
TPW = (NWIN_FULL + 1 + NW - 1) // NW
GRPS = WCOL // L
OROWS = WCOL // RPS


def _tpose_body(ut_hbm, it_hbm, utab_hbm, itab_hbm, in_v, tail_v, out_v):
    c = lax.axis_index("c")
    s = lax.axis_index("s")
    wid = s * NC + c

    lanes = lax.iota(jnp.int32, L)
    lanes16 = lanes * D
    colv = [(lanes16 + f) & (SUP - 1) for f in range(D)]
    rowof = [lax.shift_right_logical(lanes16 + f, 7) for f in range(D)]

    for src, dst in ((ut_hbm, utab_hbm), (it_hbm, itab_hbm)):
        def win(t, carry):
            w_id = wid + NW * t

            @pl.when(w_id < NWIN_FULL)
            def _full():
                c0 = w_id * WCOL
                pltpu.sync_copy(src.at[:, pl.ds(c0, WCOL)], in_v)

                def grp(g, carry2):
                    for f in range(D):
                        val = in_v[f, pl.ds(g * L, L)]
                        row = rowof[f] + g * 2
                        plsc.store_scatter(out_v, [row, colv[f]], val)
                    return carry2

                lax.fori_loop(0, GRPS, grp, 0)
                pltpu.sync_copy(out_v, dst.at[pl.ds(w_id * OROWS, OROWS)])

            @pl.when(w_id == NWIN_FULL)
            def _tail():
                c0 = NWIN_FULL * WCOL
                pltpu.sync_copy(src.at[:, pl.ds(c0, TAIL)], tail_v)

                def grp(g, carry2):
                    for f in range(D):
                        val = tail_v[f, pl.ds(g * L, L)]
                        row = rowof[f] + g * 2
                        plsc.store_scatter(out_v, [row, colv[f]], val)
                    return carry2

                lax.fori_loop(0, TAIL // L, grp, 0)
                pltpu.sync_copy(
                    out_v.at[pl.ds(0, TAIL // RPS)],
                    dst.at[pl.ds(NWIN_FULL * OROWS, TAIL // RPS)])

            return carry

        lax.fori_loop(0, TPW, win, 0)


def _gather_body(uidx_hbm, iidx_hbm, utab_hbm, itab_hbm, out_hbm,
                 uidx_v, iidx_v, usid_v, isid_v, usup_v, isup_v, out_v,
                 usem, isem):
    c = lax.axis_index("c")
    s = lax.axis_index("s")
    wid = s * NC + c

    pltpu.sync_copy(uidx_hbm.at[wid], uidx_v)
    pltpu.sync_copy(iidx_hbm.at[wid], iidx_v)

    def sid(t, carry):
        sl = pl.ds(t * L, L)
        usid_v[sl] = lax.shift_right_logical(uidx_v[sl], 3)
        isid_v[sl] = lax.shift_right_logical(iidx_v[sl], 3)
        return carry

    lax.fori_loop(0, BPW // L, sid, 0)

    lanes = lax.iota(jnp.int32, L)

    for k in range(NCHUNK):
        ucp = pltpu.async_copy(
            utab_hbm.at[usid_v.at[pl.ds(k * CHUNK, CHUNK)]], usup_v, usem)
        icp = pltpu.async_copy(
            itab_hbm.at[isid_v.at[pl.ds(k * CHUNK, CHUNK)]], isup_v, isem)
        ucp.wait()
        icp.wait()

        def group(g, carry):
            j0 = k * CHUNK + g * L
            uix = uidx_v[pl.ds(j0, L)]
            iix = iidx_v[pl.ds(j0, L)]
            ucol = (uix & 7) * D
            icol = (iix & 7) * D
            rows = lanes + g * L
            acc = jnp.zeros((L,), jnp.float32)
            for d in range(D):
                uv = plsc.load_gather(usup_v, [rows, ucol + d])
                iv = plsc.load_gather(isup_v, [rows, icol + d])
                acc = acc + uv * iv
            out_v[pl.ds(j0, L)] = 1.0 / (1.0 + jnp.exp(-acc))
            return carry

        lax.fori_loop(0, GPC, group, 0)

    pltpu.sync_copy(out_v, out_hbm.at[wid])


@jax.jit
def kernel(userIdx, itemIdx, uEmbed, iEmbed):
    uidx = userIdx.astype(jnp.int32).reshape(NW, BPW)
    iidx = itemIdx.astype(jnp.int32).reshape(NW, BPW)
    mesh = plsc.VectorSubcoreMesh(
        core_axis_name="c", subcore_axis_name="s",
        num_cores=NC, num_subcores=NS)
    utab, itab = pl.kernel(
        _tpose_body,
        out_type=(
            jax.ShapeDtypeStruct((NSUP, SUP), jnp.float32),
            jax.ShapeDtypeStruct((NSUP, SUP), jnp.float32),
        ),
        mesh=mesh,
        compiler_params=pltpu.CompilerParams(
            needs_layout_passes=False, use_tc_tiling_on_sc=True),
        scratch_types=[
            pltpu.VMEM((D, WCOL), jnp.float32),
            pltpu.VMEM((D, TAIL), jnp.float32),
            pltpu.VMEM((OROWS, SUP), jnp.float32),
        ],
    )(uEmbed.T, iEmbed.T)
    out = pl.kernel(
        _gather_body,
        out_type=jax.ShapeDtypeStruct((NW, BPW), jnp.float32),
        mesh=mesh,
        compiler_params=pltpu.CompilerParams(
            needs_layout_passes=False, use_tc_tiling_on_sc=True),
        scratch_types=[
            pltpu.VMEM((BPW,), jnp.int32),
            pltpu.VMEM((BPW,), jnp.int32),
            pltpu.VMEM((BPW,), jnp.int32),
            pltpu.VMEM((BPW,), jnp.int32),
            pltpu.VMEM((CHUNK, SUP), jnp.float32),
            pltpu.VMEM((CHUNK, SUP), jnp.float32),
            pltpu.VMEM((BPW,), jnp.float32),
            pltpu.SemaphoreType.DMA,
            pltpu.SemaphoreType.DMA,
        ],
    )(uidx, iidx, utab, itab)
    return out.reshape(-1)


# speedup vs baseline: 3.0535x; 1.4940x over previous
import jax
import jax.numpy as jnp
from jax import lax
from jax.experimental import pallas as pl
from jax.experimental.pallas import tpu as pltpu
from jax.experimental.pallas import tpu_sc as plsc

NC, NS, L = 2, 16, 16
NW = NC * NS
B = 16384
BPW = B // NW
NCHUNK = 4
CHUNK = BPW // NCHUNK
D = 16
RPS = 8
SUP = 128
GPC = CHUNK // L
N = 1000000
NSUP = N // RPS

WCOL = 896
NWIN_FULL = N // WCOL
TAIL = N - NWIN_FULL * WCOL
TPW = (NWIN_FULL + 1 + NW - 1) // NW
NPAIR = (TPW + 2) // 2
GRPS = WCOL // L
OROWS = WCOL // RPS
TAIL_WID = NWIN_FULL % NW


def _tpose_body(ut_hbm, it_hbm, utab_hbm, itab_hbm,
                in0, in1, tail_v, out0, out1,
                si0, si1, so0, so1):
    c = lax.axis_index("c")
    s = lax.axis_index("s")
    wid = s * NC + c

    lanes = lax.iota(jnp.int32, L)
    lanes16 = lanes * D
    colv = [(lanes16 + f) & (SUP - 1) for f in range(D)]
    rowof = [lax.shift_right_logical(lanes16 + f, 7) for f in range(D)]
    ins, outs = (in0, in1), (out0, out1)
    sis, sos = (si0, si1), (so0, so1)

    def shuffle(src_v, dst_v, ngrp):
        def grp(g, carry2):
            for f in range(D):
                val = src_v[f, pl.ds(g * L, L)]
                row = rowof[f] + g * 2
                plsc.store_scatter(dst_v, [row, colv[f]], val)
            return carry2

        lax.fori_loop(0, ngrp, grp, 0)

    for src, dst in ((ut_hbm, utab_hbm), (it_hbm, itab_hbm)):
        for b in range(2):
            pltpu.async_copy(
                src.at[:, pl.ds((wid + NW * b) * WCOL, WCOL)], ins[b], sis[b])

        def pair(tt, carry):
            for b in range(2):
                t = tt * 2 + b
                w_id = wid + NW * t

                @pl.when(w_id < NWIN_FULL)
                def _slot():
                    pltpu.make_async_copy(
                        src.at[:, pl.ds(w_id * WCOL, WCOL)],
                        ins[b], sis[b]).wait()

                    @pl.when(t >= 2)
                    def _wout():
                        pltpu.make_async_copy(
                            outs[b],
                            dst.at[pl.ds((w_id - 2 * NW) * OROWS, OROWS)],
                            sos[b]).wait()

                    shuffle(ins[b], outs[b], GRPS)
                    pltpu.async_copy(
                        outs[b], dst.at[pl.ds(w_id * OROWS, OROWS)], sos[b])

                    w2 = wid + NW * (t + 2)

                    @pl.when(w2 < NWIN_FULL)
                    def _pref():
                        pltpu.async_copy(
                            src.at[:, pl.ds(w2 * WCOL, WCOL)], ins[b], sis[b])

            return carry

        lax.fori_loop(0, NPAIR, pair, 0)

        for b in range(2):
            pltpu.make_async_copy(
                outs[b], dst.at[pl.ds(0, OROWS)], sos[b]).wait()

        @pl.when(wid == TAIL_WID)
        def _tail():
            pltpu.sync_copy(
                src.at[:, pl.ds(NWIN_FULL * WCOL, TAIL)], tail_v)
            shuffle(tail_v, outs[0], TAIL // L)
            pltpu.sync_copy(
                outs[0].at[pl.ds(0, TAIL // RPS)],
                dst.at[pl.ds(NWIN_FULL * OROWS, TAIL // RPS)])


def _gather_body(uidx_hbm, iidx_hbm, utab_hbm, itab_hbm, out_hbm,
                 uidx_v, iidx_v, usid_v, isid_v, usup_v, isup_v, out_v,
                 usem, isem):
    c = lax.axis_index("c")
    s = lax.axis_index("s")
    wid = s * NC + c

    pltpu.sync_copy(uidx_hbm.at[wid], uidx_v)
    pltpu.sync_copy(iidx_hbm.at[wid], iidx_v)

    def sid(t, carry):
        sl = pl.ds(t * L, L)
        usid_v[sl] = lax.shift_right_logical(uidx_v[sl], 3)
        isid_v[sl] = lax.shift_right_logical(iidx_v[sl], 3)
        return carry

    lax.fori_loop(0, BPW // L, sid, 0)

    lanes = lax.iota(jnp.int32, L)

    for k in range(NCHUNK):
        ucp = pltpu.async_copy(
            utab_hbm.at[usid_v.at[pl.ds(k * CHUNK, CHUNK)]], usup_v, usem)
        icp = pltpu.async_copy(
            itab_hbm.at[isid_v.at[pl.ds(k * CHUNK, CHUNK)]], isup_v, isem)
        ucp.wait()
        icp.wait()

        def group(g, carry):
            j0 = k * CHUNK + g * L
            uix = uidx_v[pl.ds(j0, L)]
            iix = iidx_v[pl.ds(j0, L)]
            ucol = (uix & 7) * D
            icol = (iix & 7) * D
            rows = lanes + g * L
            acc = jnp.zeros((L,), jnp.float32)
            for d in range(D):
                uv = plsc.load_gather(usup_v, [rows, ucol + d])
                iv = plsc.load_gather(isup_v, [rows, icol + d])
                acc = acc + uv * iv
            out_v[pl.ds(j0, L)] = 1.0 / (1.0 + jnp.exp(-acc))
            return carry

        lax.fori_loop(0, GPC, group, 0)

    pltpu.sync_copy(out_v, out_hbm.at[wid])


@jax.jit
def kernel(userIdx, itemIdx, uEmbed, iEmbed):
    uidx = userIdx.astype(jnp.int32).reshape(NW, BPW)
    iidx = itemIdx.astype(jnp.int32).reshape(NW, BPW)
    mesh = plsc.VectorSubcoreMesh(
        core_axis_name="c", subcore_axis_name="s",
        num_cores=NC, num_subcores=NS)
    utab, itab = pl.kernel(
        _tpose_body,
        out_type=(
            jax.ShapeDtypeStruct((NSUP, SUP), jnp.float32),
            jax.ShapeDtypeStruct((NSUP, SUP), jnp.float32),
        ),
        mesh=mesh,
        compiler_params=pltpu.CompilerParams(
            needs_layout_passes=False, use_tc_tiling_on_sc=True),
        scratch_types=[
            pltpu.VMEM((D, WCOL), jnp.float32),
            pltpu.VMEM((D, WCOL), jnp.float32),
            pltpu.VMEM((D, TAIL), jnp.float32),
            pltpu.VMEM((OROWS, SUP), jnp.float32),
            pltpu.VMEM((OROWS, SUP), jnp.float32),
            pltpu.SemaphoreType.DMA,
            pltpu.SemaphoreType.DMA,
            pltpu.SemaphoreType.DMA,
            pltpu.SemaphoreType.DMA,
        ],
    )(uEmbed.T, iEmbed.T)
    out = pl.kernel(
        _gather_body,
        out_type=jax.ShapeDtypeStruct((NW, BPW), jnp.float32),
        mesh=mesh,
        compiler_params=pltpu.CompilerParams(
            needs_layout_passes=False, use_tc_tiling_on_sc=True),
        scratch_types=[
            pltpu.VMEM((BPW,), jnp.int32),
            pltpu.VMEM((BPW,), jnp.int32),
            pltpu.VMEM((BPW,), jnp.int32),
            pltpu.VMEM((BPW,), jnp.int32),
            pltpu.VMEM((CHUNK, SUP), jnp.float32),
            pltpu.VMEM((CHUNK, SUP), jnp.float32),
            pltpu.VMEM((BPW,), jnp.float32),
            pltpu.SemaphoreType.DMA,
            pltpu.SemaphoreType.DMA,
        ],
    )(uidx, iidx, utab, itab)
    return out.reshape(-1)
